# baseline (speedup 1.0000x reference)
import jax
import jax.numpy as jnp
from jax.experimental import pallas as pl
from jax.experimental.pallas import tpu as pltpu

HEIGHT = 32
WIDTH = 32
D_MODEL = 768

B_BLK = 2


def _add_pos_body(a_ref, b_ref, row_ref, col_ref, out_ref):
    pos = (row_ref[...][:, None, :] + col_ref[...][None, :, :]).reshape(
        1, HEIGHT * WIDTH, D_MODEL
    )
    out_ref[0] = a_ref[...] + pos
    out_ref[1] = b_ref[...] + pos


def kernel(x, row_embed, col_embed):
    batch, seq_len, d = x.shape
    half = batch // 2
    steps = half // B_BLK
    out = pl.pallas_call(
        _add_pos_body,
        grid=(steps,),
        in_specs=[
            pl.BlockSpec((B_BLK, seq_len, d), lambda b: (b, 0, 0)),
            pl.BlockSpec((B_BLK, seq_len, d), lambda b, s=steps: (b + s, 0, 0)),
            pl.BlockSpec((HEIGHT, d), lambda b: (0, 0)),
            pl.BlockSpec((WIDTH, d), lambda b: (0, 0)),
        ],
        out_specs=pl.BlockSpec((2, B_BLK, seq_len, d), lambda b: (0, b, 0, 0)),
        out_shape=jax.ShapeDtypeStruct((2, half, seq_len, d), x.dtype),
        compiler_params=pltpu.CompilerParams(vmem_limit_bytes=120 * 1024 * 1024),
    )(x, x, row_embed, col_embed)
    return out.reshape(batch, seq_len, d)


# speedup vs baseline: 1.0034x; 1.0034x over previous
import jax
import jax.numpy as jnp
from jax.experimental import pallas as pl
from jax.experimental.pallas import tpu as pltpu

HEIGHT = 32
WIDTH = 32
D_MODEL = 768


B_BLK = 4


def _add_pos_body(x_ref, row_ref, col_ref, out_ref):
    pos = (row_ref[...][:, None, :] + col_ref[...][None, :, :]).reshape(
        1, HEIGHT * WIDTH, D_MODEL
    )
    out_ref[...] = x_ref[...] + pos


def kernel(x, row_embed, col_embed):
    batch, seq_len, d = x.shape
    out = pl.pallas_call(
        _add_pos_body,
        grid=(batch // B_BLK,),
        in_specs=[
            pl.BlockSpec((B_BLK, seq_len, d), lambda b: (b, 0, 0)),
            pl.BlockSpec((HEIGHT, d), lambda b: (0, 0)),
            pl.BlockSpec((WIDTH, d), lambda b: (0, 0)),
        ],
        out_specs=pl.BlockSpec((B_BLK, seq_len, d), lambda b: (b, 0, 0)),
        out_shape=jax.ShapeDtypeStruct((batch, seq_len, d), x.dtype),
        compiler_params=pltpu.CompilerParams(vmem_limit_bytes=120 * 1024 * 1024),
    )(x, row_embed, col_embed)
    return out
